# baseline (speedup 1.0000x reference)
import jax
import jax.numpy as jnp
from jax.experimental import pallas as pl
from jax.experimental.pallas import tpu as pltpu

_T = 0.05
_KTILE = 4096


def _mm_kernel(x_ref, m_ref, o_ref):
    o_ref[...] = jax.lax.dot_general(
        x_ref[...] / _T,
        m_ref[...],
        dimension_numbers=(((1,), (1,)), ((), ())),
        preferred_element_type=jnp.float32,
    )


def kernel(x, mem_feat):
    q, d = x.shape
    k = mem_feat.shape[0]
    grid = (k // _KTILE,)
    return pl.pallas_call(
        _mm_kernel,
        grid=grid,
        in_specs=[
            pl.BlockSpec((q, d), lambda i: (0, 0)),
            pl.BlockSpec((_KTILE, d), lambda i: (i, 0)),
        ],
        out_specs=pl.BlockSpec((q, _KTILE), lambda i: (0, i)),
        out_shape=jax.ShapeDtypeStruct((q, k), jnp.float32),
        compiler_params=pltpu.CompilerParams(
            dimension_semantics=("arbitrary",),
        ),
    )(x, mem_feat)


# speedup vs baseline: 1.0016x; 1.0016x over previous
import jax
import jax.numpy as jnp
from jax.experimental import pallas as pl
from jax.experimental.pallas import tpu as pltpu

_T = 0.05
_KTILE = 4096


def _mm_kernel(x_ref, m_ref, o_ref):
    o_ref[...] = jax.lax.dot_general(
        x_ref[...] / _T,
        m_ref[...],
        dimension_numbers=(((1,), (1,)), ((), ())),
        preferred_element_type=jnp.float32,
    )


def kernel(x, mem_feat):
    q, d = x.shape
    k = mem_feat.shape[0]
    grid = (k // _KTILE,)
    return pl.pallas_call(
        _mm_kernel,
        grid=grid,
        in_specs=[
            pl.BlockSpec((q, d), lambda i: (0, 0)),
            pl.BlockSpec((_KTILE, d), lambda i: (i, 0)),
        ],
        out_specs=pl.BlockSpec((q, _KTILE), lambda i: (0, i)),
        out_shape=jax.ShapeDtypeStruct((q, k), jnp.float32),
        compiler_params=pltpu.CompilerParams(
            dimension_semantics=("parallel",),
        ),
    )(x, mem_feat)
